# Initial kernel scaffold; baseline (speedup 1.0000x reference)
import jax
import jax.numpy as jnp
from jax.experimental import pallas as pl


def kernel(x, edge_index, batchs, W1, as1, ad1, b1, Wg, asg, adg, bg, Wl1, bl1, Wls, bls, Wl3, bl3):
    raise NotImplementedError("write your pallas kernel here")


# speedup vs baseline: 1.5003x; 1.5003x over previous
import jax
import jax.numpy as jnp
from jax.experimental import pallas as pl
from jax.experimental.pallas import tpu as pltpu

N = 10000
E = 320000
D = 128
G = 64
C = 2
N_GRAPH_LAYER = 2
N_FC_LAYER = 2


def _gat_layer(x, W, a_src, a_dst, b, src, dst):
    h = x @ W
    alpha_s = h @ a_src
    alpha_d = h @ a_dst
    e = jax.nn.leaky_relu(alpha_s[src] + alpha_d[dst], negative_slope=0.2)
    ex = jnp.exp(e)
    denom = jax.ops.segment_sum(ex, dst, num_segments=N)
    num = jax.ops.segment_sum(ex[:, None] * h[src], dst, num_segments=N)
    return num / (denom[:, None] + 1e-16) + b


def _head_kernel(h_ref, batchs_ref, Wl1_ref, bl1_ref, Wls_ref, bls_ref,
                 Wl3_ref, bl3_ref, out_ref):
    h = h_ref[...]
    batchs = batchs_ref[...]
    gids = jax.lax.broadcasted_iota(jnp.int32, (G, N), 0)
    onehot = (gids == batchs[None, :]).astype(jnp.float32)
    sums = jnp.dot(onehot, h, preferred_element_type=jnp.float32)
    cnt = jnp.sum(onehot, axis=1)
    p = sums / jnp.clip(cnt, 1.0)[:, None]
    p = jax.nn.relu(jnp.dot(p, Wl1_ref[...], preferred_element_type=jnp.float32) + bl1_ref[...])
    for i in range(N_FC_LAYER):
        p = jax.nn.relu(jnp.dot(p, Wls_ref[i], preferred_element_type=jnp.float32) + bls_ref[i])
    p = jnp.dot(p, Wl3_ref[...], preferred_element_type=jnp.float32) + bl3_ref[...]
    m = jnp.max(p, axis=1, keepdims=True)
    lse = jnp.log(jnp.sum(jnp.exp(p - m), axis=1, keepdims=True)) + m
    out_ref[...] = p - lse


def kernel(x, edge_index, batchs, W1, as1, ad1, b1, Wg, asg, adg, bg, Wl1, bl1, Wls, bls, Wl3, bl3):
    loop = jnp.arange(N, dtype=edge_index.dtype)
    src = jnp.concatenate([edge_index[0], loop])
    dst = jnp.concatenate([edge_index[1], loop])
    h = _gat_layer(x, W1, as1, ad1, b1, src, dst)
    h = jax.nn.relu(h)
    for i in range(N_GRAPH_LAYER):
        h = _gat_layer(h, Wg[i], asg[i], adg[i], bg[i], src, dst)
        h = jax.nn.relu(h)
    out = pl.pallas_call(
        _head_kernel,
        out_shape=jax.ShapeDtypeStruct((G, C), jnp.float32),
    )(h, batchs, Wl1, bl1, Wls, bls, Wl3, bl3)
    return out


# speedup vs baseline: 21.9217x; 14.6118x over previous
import functools

import jax
import jax.numpy as jnp
from jax import lax
from jax.experimental import pallas as pl
from jax.experimental.pallas import tpu as pltpu
from jax.experimental.pallas import tpu_sc as plsc

N = 10000
E = 320000
D = 128
G = 64
C = 2
N_GRAPH_LAYER = 2
N_FC_LAYER = 2

E2 = E + N
NW = 32
SB = 128
NB = 81
EPW = NB * SB
E_PAD = NW * EPW
NPAD = 10240
STRIPE = NPAD // 16

_MESH = plsc.VectorSubcoreMesh(
    core_axis_name="c", subcore_axis_name="s", num_cores=2, num_subcores=16)


RSTRIPE = N // 16


def _edge_body(as_hbm, ad_hbm, src_hbm, dst_hbm, h_hbm,
               acc_out, den_out,
               as_v, ad_v, srcb, dstb, ex_buf, rows, zbuf, zd,
               acc, den_acc, sem):
    cid = lax.axis_index("c")
    sid = lax.axis_index("s")
    wid = sid * 2 + cid

    pltpu.sync_copy(as_hbm, as_v)
    pltpu.sync_copy(ad_hbm, ad_v)

    zeros16 = jnp.zeros((16,), jnp.float32)

    def _zb(i, _):
        zbuf[i // 8, pl.ds((i % 8) * 16, 16)] = zeros16
        return 0
    lax.fori_loop(0, 25 * 8, _zb, 0)

    def _zd(i, _):
        zd[pl.ds(i * 16, 16)] = zeros16
        return 0
    lax.fori_loop(0, STRIPE // 16, _zd, 0)

    for k in range(RSTRIPE // 25):
        pltpu.sync_copy(zbuf, acc.at[pl.ds(sid * RSTRIPE + k * 25, 25)])
    pltpu.sync_copy(zd, den_acc.at[pl.ds(sid * STRIPE, STRIPE)])
    plsc.subcore_barrier()

    ebase = wid * EPW

    def _batch(j, _):
        pltpu.sync_copy(src_hbm.at[wid].at[j], srcb.at[0])
        pltpu.sync_copy(dst_hbm.at[wid].at[j], dstb.at[0])
        cp = pltpu.async_copy(h_hbm.at[srcb.at[0]], rows, sem)
        for g in range(8):
            sl = pl.ds(g * 16, 16)
            sv = srcb[0, sl]
            dv = dstb[0, sl]
            es = plsc.load_gather(as_v, [sv])
            ed = plsc.load_gather(ad_v, [dv])
            e = es + ed
            e = jnp.maximum(e, 0.2 * e)
            ex = jnp.exp(e)
            gidx = (ebase + j * SB + g * 16
                    + lax.broadcasted_iota(jnp.int32, (16,), 0))
            ex = jnp.where(gidx < E2, ex, 0.0)
            ex_buf[sl] = ex
        cp.wait()

        def _scale(i, _):
            exv = plsc.load_gather(
                ex_buf, [jnp.broadcast_to(i, (16,)).astype(jnp.int32)])
            for kk in range(8):
                sl = pl.ds(kk * 16, 16)
                rows[i, sl] = rows[i, sl] * exv
            return 0
        lax.fori_loop(0, SB, _scale, 0)

        pltpu.sync_copy(rows, acc.at[dstb.at[0]], add=True)
        pltpu.sync_copy(ex_buf, den_acc.at[dstb.at[0]], add=True)
        return 0

    lax.fori_loop(0, NB, _batch, 0)
    plsc.subcore_barrier()

    pltpu.sync_copy(acc.at[pl.ds(sid * RSTRIPE, RSTRIPE)],
                    acc_out.at[cid, sid])
    pltpu.sync_copy(den_acc.at[pl.ds(sid * STRIPE, STRIPE)],
                    den_out.at[cid, sid])


_edge_kernel = functools.partial(
    pl.kernel,
    out_type=[
        jax.ShapeDtypeStruct((2, 16, RSTRIPE, D), jnp.float32),
        jax.ShapeDtypeStruct((2, 16, STRIPE), jnp.float32),
    ],
    mesh=_MESH,
    compiler_params=pltpu.CompilerParams(needs_layout_passes=False),
    scratch_types=[
        pltpu.VMEM((N,), jnp.float32),
        pltpu.VMEM((N,), jnp.float32),
        pltpu.VMEM((1, SB), jnp.int32),
        pltpu.VMEM((1, SB), jnp.int32),
        pltpu.VMEM((SB,), jnp.float32),
        pltpu.VMEM((SB, D), jnp.float32),
        pltpu.VMEM((25, D), jnp.float32),
        pltpu.VMEM((STRIPE,), jnp.float32),
        pltpu.VMEM_SHARED((N, D), jnp.float32),
        pltpu.VMEM_SHARED((NPAD,), jnp.float32),
        pltpu.SemaphoreType.DMA,
    ],
)(_edge_body)


def _mm_first_body(x_ref, W_ref, as_ref, ad_ref, h_ref, asv_ref, adv_ref):
    h = jnp.dot(x_ref[...], W_ref[...], preferred_element_type=jnp.float32)
    h_ref[...] = h
    asv_ref[...] = jnp.dot(h, as_ref[...], preferred_element_type=jnp.float32)
    adv_ref[...] = jnp.dot(h, ad_ref[...], preferred_element_type=jnp.float32)


def _mm_combine_body(accp_ref, denp_ref, b_ref, W_ref, as_ref, ad_ref,
                     h_ref, asv_ref, adv_ref):
    num = accp_ref[0, :N, :] + accp_ref[1, :N, :]
    den = denp_ref[0, :N] + denp_ref[1, :N]
    hprev = jax.nn.relu(num / (den[:, None] + 1e-16) + b_ref[...])
    h = jnp.dot(hprev, W_ref[...], preferred_element_type=jnp.float32)
    h_ref[...] = h
    asv_ref[...] = jnp.dot(h, as_ref[...], preferred_element_type=jnp.float32)
    adv_ref[...] = jnp.dot(h, ad_ref[...], preferred_element_type=jnp.float32)


def _head_body(accp_ref, denp_ref, b_ref, batchs_ref,
               Wl1_ref, bl1_ref, Wls_ref, bls_ref, Wl3_ref, bl3_ref, out_ref):
    num = accp_ref[0, :N, :] + accp_ref[1, :N, :]
    den = denp_ref[0, :N] + denp_ref[1, :N]
    h = jax.nn.relu(num / (den[:, None] + 1e-16) + b_ref[...])
    batchs = batchs_ref[...]
    gids = lax.broadcasted_iota(jnp.int32, (G, N), 0)
    onehot = (gids == batchs[None, :]).astype(jnp.float32)
    sums = jnp.dot(onehot, h, preferred_element_type=jnp.float32)
    cnt = jnp.sum(onehot, axis=1)
    p = sums / jnp.clip(cnt, 1.0)[:, None]
    p = jax.nn.relu(jnp.dot(p, Wl1_ref[...],
                            preferred_element_type=jnp.float32) + bl1_ref[...])
    for i in range(N_FC_LAYER):
        p = jax.nn.relu(jnp.dot(p, Wls_ref[i],
                                preferred_element_type=jnp.float32) + bls_ref[i])
    p = jnp.dot(p, Wl3_ref[...], preferred_element_type=jnp.float32) + bl3_ref[...]
    m = jnp.max(p, axis=1, keepdims=True)
    lse = jnp.log(jnp.sum(jnp.exp(p - m), axis=1, keepdims=True)) + m
    out_ref[...] = p - lse


def _mm_first(x, W, a_s, a_d):
    return pl.pallas_call(
        _mm_first_body,
        out_shape=[
            jax.ShapeDtypeStruct((N, D), jnp.float32),
            jax.ShapeDtypeStruct((N, 1), jnp.float32),
            jax.ShapeDtypeStruct((N, 1), jnp.float32),
        ],
    )(x, W, a_s.reshape(D, 1), a_d.reshape(D, 1))


def _mm_combine(accp, denp, b, W, a_s, a_d):
    return pl.pallas_call(
        _mm_combine_body,
        out_shape=[
            jax.ShapeDtypeStruct((N, D), jnp.float32),
            jax.ShapeDtypeStruct((N, 1), jnp.float32),
            jax.ShapeDtypeStruct((N, 1), jnp.float32),
        ],
    )(accp.reshape(2, N, D), denp.reshape(2, NPAD), b, W,
      a_s.reshape(D, 1), a_d.reshape(D, 1))


def _head(accp, denp, b, batchs, Wl1, bl1, Wls, bls, Wl3, bl3):
    return pl.pallas_call(
        _head_body,
        out_shape=jax.ShapeDtypeStruct((G, C), jnp.float32),
    )(accp.reshape(2, N, D), denp.reshape(2, NPAD), b, batchs,
      Wl1, bl1, Wls, bls, Wl3, bl3)


def kernel(x, edge_index, batchs, W1, as1, ad1, b1, Wg, asg, adg, bg,
           Wl1, bl1, Wls, bls, Wl3, bl3):
    loop = jnp.arange(N, dtype=edge_index.dtype)
    pad = jnp.zeros((E_PAD - E2,), edge_index.dtype)
    src = jnp.concatenate([edge_index[0], loop, pad]).reshape(NW, NB, SB)
    dst = jnp.concatenate([edge_index[1], loop, pad]).reshape(NW, NB, SB)

    h, asv, adv = _mm_first(x, W1, as1, ad1)
    accp, denp = _edge_kernel(asv.reshape(N), adv.reshape(N), src, dst, h)
    for i in range(N_GRAPH_LAYER):
        h, asv, adv = _mm_combine(accp, denp, b1 if i == 0 else bg[i - 1],
                                  Wg[i], asg[i], adg[i])
        accp, denp = _edge_kernel(asv.reshape(N), adv.reshape(N), src, dst, h)
    return _head(accp, denp, bg[N_GRAPH_LAYER - 1], batchs,
                 Wl1, bl1, Wls, bls, Wl3, bl3)


# speedup vs baseline: 30.2181x; 1.3785x over previous
import functools

import jax
import jax.numpy as jnp
from jax import lax
from jax.experimental import pallas as pl
from jax.experimental.pallas import tpu as pltpu
from jax.experimental.pallas import tpu_sc as plsc

N = 10000
E = 320000
D = 128
G = 64
C = 2
N_GRAPH_LAYER = 2
N_FC_LAYER = 2

E2 = E + N
NW = 32
SB = 96
NPAIR = 54
NB = 2 * NPAIR
EPW = NB * SB
E_PAD = NW * EPW
NPAD = 10240
STRIPE = NPAD // 16

_MESH = plsc.VectorSubcoreMesh(
    core_axis_name="c", subcore_axis_name="s", num_cores=2, num_subcores=16)


RSTRIPE = N // 16


def _edge_body(as_hbm, ad_hbm, sd_hbm, h_hbm,
               acc_out, den_out,
               as_v, ad_v, ib, exA, exB, rowsA, rowsB,
               acc, den_acc, semA, semB):
    cid = lax.axis_index("c")
    sid = lax.axis_index("s")
    wid = sid * 2 + cid

    pltpu.sync_copy(as_hbm, as_v)
    pltpu.sync_copy(ad_hbm, ad_v)

    zeros16 = jnp.zeros((16,), jnp.float32)

    def _zb(i, _):
        rowsA[i // 8, pl.ds((i % 8) * 16, 16)] = zeros16
        return 0
    lax.fori_loop(0, SB * 8, _zb, 0)

    for k in range(RSTRIPE // 25):
        pltpu.sync_copy(rowsA.at[pl.ds(0, 25)],
                        acc.at[pl.ds(sid * RSTRIPE + k * 25, 25)])
    for k in range(STRIPE // 128):
        pltpu.sync_copy(rowsA.at[0, pl.ds(0, 128)],
                        den_acc.at[pl.ds(sid * STRIPE + k * 128, 128)])
    plsc.subcore_barrier()

    ebase = wid * EPW
    iota16 = lax.broadcasted_iota(jnp.int32, (16,), 0)

    def _comp_ex(ibuf_pb, a, jbase, ex_buf):
        for g in range(SB // 16):
            sl = pl.ds(g * 16, 16)
            sv = ibuf_pb[a, 0, sl]
            dv = ibuf_pb[a, 1, sl]
            es = plsc.load_gather(as_v, [sv])
            ed = plsc.load_gather(ad_v, [dv])
            e = es + ed
            e = jnp.maximum(e, 0.2 * e)
            ex = jnp.exp(e)
            gidx = ebase + jbase * SB + g * 16 + iota16
            ex_buf[sl] = jnp.where(gidx < E2, ex, 0.0)

    def _scale_scatter(rows, ex_buf, dst_idx):
        def _scale(i, _):
            exv = plsc.load_gather(
                ex_buf, [jnp.broadcast_to(i, (16,)).astype(jnp.int32)])
            for kk in range(8):
                sl = pl.ds(kk * 16, 16)
                rows[i, sl] = rows[i, sl] * exv
            return 0
        lax.fori_loop(0, SB, _scale, 0)
        pltpu.sync_copy(rows, acc.at[dst_idx], add=True)
        pltpu.sync_copy(ex_buf, den_acc.at[dst_idx], add=True)

    pltpu.sync_copy(sd_hbm.at[wid, 0], ib.at[0])
    gA = pltpu.async_copy(h_hbm.at[ib.at[0, 0, 0]], rowsA, semA)

    def _pair(jj, _):
        pb = jax.lax.rem(jj, 2)
        nb = 1 - pb
        ipb = ib.at[pb]
        pltpu.async_copy(h_hbm.at[ipb.at[1, 0]], rowsB, semB)
        @pl.when(jj < NPAIR - 1)
        def _():
            pltpu.sync_copy(sd_hbm.at[wid, jj + 1], ib.at[nb])
        _comp_ex(ipb, 0, 2 * jj, exA)
        pltpu.make_async_copy(h_hbm.at[ipb.at[0, 0]], rowsA, semA).wait()
        _scale_scatter(rowsA, exA, ipb.at[0, 1])
        _comp_ex(ipb, 1, 2 * jj + 1, exB)
        pltpu.make_async_copy(h_hbm.at[ipb.at[1, 0]], rowsB, semB).wait()

        @pl.when(jj < NPAIR - 1)
        def _():
            pltpu.async_copy(h_hbm.at[ib.at[nb, 0, 0]], rowsA, semA)
        _scale_scatter(rowsB, exB, ipb.at[1, 1])
        return 0

    lax.fori_loop(0, NPAIR, _pair, 0)
    plsc.subcore_barrier()

    pltpu.sync_copy(acc.at[pl.ds(sid * RSTRIPE, RSTRIPE)],
                    acc_out.at[cid, sid])
    pltpu.sync_copy(den_acc.at[pl.ds(sid * STRIPE, STRIPE)],
                    den_out.at[cid, sid])


_edge_kernel = functools.partial(
    pl.kernel,
    out_type=[
        jax.ShapeDtypeStruct((2, 16, RSTRIPE, D), jnp.float32),
        jax.ShapeDtypeStruct((2, 16, STRIPE), jnp.float32),
    ],
    mesh=_MESH,
    compiler_params=pltpu.CompilerParams(needs_layout_passes=False),
    scratch_types=[
        pltpu.VMEM((N,), jnp.float32),
        pltpu.VMEM((N,), jnp.float32),
        pltpu.VMEM((2, 2, 2, SB), jnp.int32),
        pltpu.VMEM((SB,), jnp.float32),
        pltpu.VMEM((SB,), jnp.float32),
        pltpu.VMEM((SB, D), jnp.float32),
        pltpu.VMEM((SB, D), jnp.float32),
        pltpu.VMEM_SHARED((N, D), jnp.float32),
        pltpu.VMEM_SHARED((NPAD,), jnp.float32),
        pltpu.SemaphoreType.DMA,
        pltpu.SemaphoreType.DMA,
    ],
)(_edge_body)


def _mm_first_body(x_ref, W_ref, as_ref, ad_ref, h_ref, asv_ref, adv_ref):
    h = jnp.dot(x_ref[...], W_ref[...], preferred_element_type=jnp.float32)
    h_ref[...] = h
    asv_ref[...] = jnp.dot(h, as_ref[...], preferred_element_type=jnp.float32)
    adv_ref[...] = jnp.dot(h, ad_ref[...], preferred_element_type=jnp.float32)


def _mm_combine_body(accp_ref, denp_ref, b_ref, W_ref, as_ref, ad_ref,
                     h_ref, asv_ref, adv_ref):
    num = accp_ref[0, :N, :] + accp_ref[1, :N, :]
    den = denp_ref[0, :N] + denp_ref[1, :N]
    hprev = jax.nn.relu(num / (den[:, None] + 1e-16) + b_ref[...])
    h = jnp.dot(hprev, W_ref[...], preferred_element_type=jnp.float32)
    h_ref[...] = h
    asv_ref[...] = jnp.dot(h, as_ref[...], preferred_element_type=jnp.float32)
    adv_ref[...] = jnp.dot(h, ad_ref[...], preferred_element_type=jnp.float32)


def _head_body(accp_ref, denp_ref, b_ref, batchs_ref,
               Wl1_ref, bl1_ref, Wls_ref, bls_ref, Wl3_ref, bl3_ref, out_ref):
    num = accp_ref[0, :N, :] + accp_ref[1, :N, :]
    den = denp_ref[0, :N] + denp_ref[1, :N]
    h = jax.nn.relu(num / (den[:, None] + 1e-16) + b_ref[...])
    batchs = batchs_ref[...]
    gids = lax.broadcasted_iota(jnp.int32, (G, N), 0)
    onehot = (gids == batchs[None, :]).astype(jnp.float32)
    sums = jnp.dot(onehot, h, preferred_element_type=jnp.float32)
    cnt = jnp.sum(onehot, axis=1)
    p = sums / jnp.clip(cnt, 1.0)[:, None]
    p = jax.nn.relu(jnp.dot(p, Wl1_ref[...],
                            preferred_element_type=jnp.float32) + bl1_ref[...])
    for i in range(N_FC_LAYER):
        p = jax.nn.relu(jnp.dot(p, Wls_ref[i],
                                preferred_element_type=jnp.float32) + bls_ref[i])
    p = jnp.dot(p, Wl3_ref[...], preferred_element_type=jnp.float32) + bl3_ref[...]
    m = jnp.max(p, axis=1, keepdims=True)
    lse = jnp.log(jnp.sum(jnp.exp(p - m), axis=1, keepdims=True)) + m
    out_ref[...] = p - lse


def _mm_first(x, W, a_s, a_d):
    return pl.pallas_call(
        _mm_first_body,
        out_shape=[
            jax.ShapeDtypeStruct((N, D), jnp.float32),
            jax.ShapeDtypeStruct((N, 1), jnp.float32),
            jax.ShapeDtypeStruct((N, 1), jnp.float32),
        ],
    )(x, W, a_s.reshape(D, 1), a_d.reshape(D, 1))


def _mm_combine(accp, denp, b, W, a_s, a_d):
    return pl.pallas_call(
        _mm_combine_body,
        out_shape=[
            jax.ShapeDtypeStruct((N, D), jnp.float32),
            jax.ShapeDtypeStruct((N, 1), jnp.float32),
            jax.ShapeDtypeStruct((N, 1), jnp.float32),
        ],
    )(accp.reshape(2, N, D), denp.reshape(2, NPAD), b, W,
      a_s.reshape(D, 1), a_d.reshape(D, 1))


def _head(accp, denp, b, batchs, Wl1, bl1, Wls, bls, Wl3, bl3):
    return pl.pallas_call(
        _head_body,
        out_shape=jax.ShapeDtypeStruct((G, C), jnp.float32),
    )(accp.reshape(2, N, D), denp.reshape(2, NPAD), b, batchs,
      Wl1, bl1, Wls, bls, Wl3, bl3)


def kernel(x, edge_index, batchs, W1, as1, ad1, b1, Wg, asg, adg, bg,
           Wl1, bl1, Wls, bls, Wl3, bl3):
    loop = jnp.arange(N, dtype=edge_index.dtype)
    pad = jnp.zeros((E_PAD - E2,), edge_index.dtype)
    src = jnp.concatenate([edge_index[0], loop, pad]).reshape(NW, NPAIR, 2, 1, SB)
    dst = jnp.concatenate([edge_index[1], loop, pad]).reshape(NW, NPAIR, 2, 1, SB)
    sd = jnp.concatenate([src, dst], axis=3)

    h, asv, adv = _mm_first(x, W1, as1, ad1)
    accp, denp = _edge_kernel(asv.reshape(N), adv.reshape(N), sd, h)
    for i in range(N_GRAPH_LAYER):
        h, asv, adv = _mm_combine(accp, denp, b1 if i == 0 else bg[i - 1],
                                  Wg[i], asg[i], adg[i])
        accp, denp = _edge_kernel(asv.reshape(N), adv.reshape(N), sd, h)
    return _head(accp, denp, bg[N_GRAPH_LAYER - 1], batchs,
                 Wl1, bl1, Wls, bls, Wl3, bl3)
